# baseline (speedup 1.0000x reference)
import functools

import jax
import jax.numpy as jnp
from jax import lax
from jax.experimental import pallas as pl
from jax.experimental.pallas import tpu as pltpu
from jax.experimental.pallas import tpu_sc as plsc

VOCAB = 100000
HIDDEN = 768
NUM_TABLES = 3
B = 4 * 2048
NC, NS = 2, 16
NW = NC * NS
B_PER_W = B // NW
CHUNK = 64
NCHUNK = B_PER_W // CHUNK
DEPTH = 2
GLEAD = 1


@jax.jit
def _gather3(idx_flat, W0, W1, W2):
    out = jax.ShapeDtypeStruct((B, HIDDEN), jnp.float32)
    mesh = plsc.VectorSubcoreMesh(core_axis_name="c", subcore_axis_name="s")

    @functools.partial(
        pl.kernel,
        out_type=(out,) * 6,
        mesh=mesh,
        scratch_types=[
            *[pltpu.VMEM((CHUNK,), jnp.int32) for _ in range(NCHUNK)],
            *[pltpu.VMEM((CHUNK, HIDDEN), jnp.float32) for _ in range(DEPTH)],
            *[pltpu.SemaphoreType.DMA for _ in range(2 * DEPTH + 1)],
        ],
    )
    def k(idx_hbm, w0_hbm, w1_hbm, w2_hbm,
          o0_hbm, o1_hbm, o2_hbm, o3_hbm, o4_hbm, o5_hbm, *scratch):
        idxs = scratch[:NCHUNK]
        bufs = scratch[NCHUNK:NCHUNK + DEPTH]
        gsems = scratch[NCHUNK + DEPTH:NCHUNK + 2 * DEPTH]
        wsems = scratch[NCHUNK + 2 * DEPTH:NCHUNK + 3 * DEPTH]
        dsem = scratch[NCHUNK + 3 * DEPTH]
        tables = (w0_hbm, w1_hbm, w2_hbm)
        outs = (o0_hbm, o1_hbm, o2_hbm, o3_hbm, o4_hbm, o5_hbm)

        wid = lax.axis_index("s") * NC + lax.axis_index("c")
        base = wid * B_PER_W

        for c in range(NCHUNK):
            pltpu.sync_copy(idx_hbm.at[pl.ds(base + c * CHUNK, CHUNK)], idxs[c])

        items = [(t, c) for t in range(NUM_TABLES) for c in range(NCHUNK)]
        n = len(items)
        gcopies, wcopies, dcopies = {}, {}, {}

        def gstart(m):
            t, c = items[m]
            gcopies[m] = pltpu.async_copy(
                tables[t].at[idxs[c]], bufs[m % DEPTH], gsems[m % DEPTH])

        def wstart(m):
            t, c = items[m]
            sl = pl.ds(base + c * CHUNK, CHUNK)
            wcopies[m] = pltpu.async_copy(bufs[m % DEPTH], outs[t].at[sl],
                                          wsems[m % DEPTH])

        def dstart(m):
            t, c = items[m]
            sl = pl.ds(base + c * CHUNK, CHUNK)
            dcopies[m] = pltpu.async_copy(outs[t].at[sl], outs[5 - t].at[sl],
                                          dsem)

        for m in range(min(GLEAD, n)):
            gstart(m)
        for j in range(n):
            gcopies[j].wait()
            wstart(j)
            m = j + GLEAD
            if m < n:
                if m >= DEPTH:
                    wcopies[m - DEPTH].wait()
                    dstart(m - DEPTH)
                gstart(m)
        for m in range(max(0, n - DEPTH), n):
            wcopies[m].wait()
            dstart(m)
        for m in range(n):
            dcopies[m].wait()

    return k(idx_flat, W0, W1, W2)


def kernel(inputs, W0, W1, W2):
    idx = inputs.reshape(-1).astype(jnp.int32)
    outs = _gather3(idx, W0, W1, W2)
    shp = (*inputs.shape, HIDDEN)
    return tuple(o.reshape(shp) for o in outs)


# speedup vs baseline: 17.8391x; 17.8391x over previous
import functools

import jax
import jax.numpy as jnp
from jax import lax
from jax.experimental import pallas as pl
from jax.experimental.pallas import tpu as pltpu
from jax.experimental.pallas import tpu_sc as plsc

VOCAB = 100000
HIDDEN = 768
B = 4 * 2048
NC, NS = 2, 16
NW = NC * NS
B_PER_W = B // NW
CHUNK = 64
NCHUNK = B_PER_W // CHUNK
DEPTH = 2
GLEAD = 1
DUP_BLOCK = 1024


def _gather1(idx_flat, W):
    mesh = plsc.VectorSubcoreMesh(core_axis_name="c", subcore_axis_name="s")

    @functools.partial(
        pl.kernel,
        out_type=jax.ShapeDtypeStruct((B, HIDDEN), jnp.float32),
        mesh=mesh,
        scratch_types=[
            *[pltpu.VMEM((CHUNK,), jnp.int32) for _ in range(NCHUNK)],
            *[pltpu.VMEM((CHUNK, HIDDEN), jnp.float32) for _ in range(DEPTH)],
            *[pltpu.SemaphoreType.DMA for _ in range(2 * DEPTH)],
        ],
    )
    def k(idx_hbm, w_hbm, o_hbm, *scratch):
        idxs = scratch[:NCHUNK]
        bufs = scratch[NCHUNK:NCHUNK + DEPTH]
        gsems = scratch[NCHUNK + DEPTH:NCHUNK + 2 * DEPTH]
        wsems = scratch[NCHUNK + 2 * DEPTH:]

        wid = lax.axis_index("s") * NC + lax.axis_index("c")
        base = wid * B_PER_W

        for c in range(NCHUNK):
            pltpu.sync_copy(idx_hbm.at[pl.ds(base + c * CHUNK, CHUNK)], idxs[c])

        n = NCHUNK
        gcopies, wcopies = {}, {}

        def gstart(m):
            gcopies[m] = pltpu.async_copy(
                w_hbm.at[idxs[m]], bufs[m % DEPTH], gsems[m % DEPTH])

        def wstart(m):
            wcopies[m] = pltpu.async_copy(
                bufs[m % DEPTH],
                o_hbm.at[pl.ds(base + m * CHUNK, CHUNK)],
                wsems[m % DEPTH])

        for m in range(min(GLEAD, n)):
            gstart(m)
        for j in range(n):
            gcopies[j].wait()
            wstart(j)
            m = j + GLEAD
            if m < n:
                if m >= DEPTH:
                    wcopies[m - DEPTH].wait()
                gstart(m)
        for m in range(max(0, n - DEPTH), n):
            wcopies[m].wait()

    return k(idx_flat, W)


def _dup_body(x_ref, o_ref):
    o_ref[...] = x_ref[...]


def _dup(x):
    return pl.pallas_call(
        _dup_body,
        out_shape=jax.ShapeDtypeStruct(x.shape, x.dtype),
        grid=(B // DUP_BLOCK,),
        in_specs=[pl.BlockSpec((DUP_BLOCK, HIDDEN), lambda i: (i, 0))],
        out_specs=pl.BlockSpec((DUP_BLOCK, HIDDEN), lambda i: (i, 0)),
    )(x)


@jax.jit
def _run(idx_flat, W0, W1, W2):
    e0 = _gather1(idx_flat, W0)
    e1 = _gather1(idx_flat, W1)
    e2 = _gather1(idx_flat, W2)
    e5 = _dup(e0)
    e4 = _dup(e1)
    e3 = _dup(e2)
    return e0, e1, e2, e3, e4, e5


def kernel(inputs, W0, W1, W2):
    idx = inputs.reshape(-1).astype(jnp.int32)
    outs = _run(idx, W0, W1, W2)
    shp = (*inputs.shape, HIDDEN)
    return tuple(o.reshape(shp) for o in outs)


# speedup vs baseline: 19.4339x; 1.0894x over previous
import functools

import jax
import jax.numpy as jnp
from jax import lax
from jax.experimental import pallas as pl
from jax.experimental.pallas import tpu as pltpu
from jax.experimental.pallas import tpu_sc as plsc

VOCAB = 100000
HIDDEN = 768
B = 4 * 2048
NC, NS = 2, 16
NW = NC * NS
B_PER_W = B // NW
CHUNK = 64
NCHUNK = B_PER_W // CHUNK
DEPTH = 2
GLEAD = 1
DUP_BLOCK = 1024


def _sc_gather(idx_flat, tables_list, n_writes):
    nt = len(tables_list)
    out = jax.ShapeDtypeStruct((B, HIDDEN), jnp.float32)
    mesh = plsc.VectorSubcoreMesh(core_axis_name="c", subcore_axis_name="s")

    @functools.partial(
        pl.kernel,
        out_type=(out,) * (nt * n_writes),
        mesh=mesh,
        scratch_types=[
            *[pltpu.VMEM((CHUNK,), jnp.int32) for _ in range(NCHUNK)],
            *[pltpu.VMEM((CHUNK, HIDDEN), jnp.float32) for _ in range(DEPTH)],
            *[pltpu.SemaphoreType.DMA for _ in range(2 * DEPTH)],
        ],
    )
    def k(idx_hbm, *rest):
        tables = rest[:nt]
        outs = rest[nt:nt + nt * n_writes]
        scratch = rest[nt + nt * n_writes:]
        idxs = scratch[:NCHUNK]
        bufs = scratch[NCHUNK:NCHUNK + DEPTH]
        gsems = scratch[NCHUNK + DEPTH:NCHUNK + 2 * DEPTH]
        wsems = scratch[NCHUNK + 2 * DEPTH:]

        wid = lax.axis_index("s") * NC + lax.axis_index("c")
        base = wid * B_PER_W

        for c in range(NCHUNK):
            pltpu.sync_copy(idx_hbm.at[pl.ds(base + c * CHUNK, CHUNK)], idxs[c])

        items = [(t, c) for t in range(nt) for c in range(NCHUNK)]
        n = len(items)
        gcopies, wcopies = {}, {}

        def gstart(m):
            t, c = items[m]
            gcopies[m] = pltpu.async_copy(
                tables[t].at[idxs[c]], bufs[m % DEPTH], gsems[m % DEPTH])

        def wstart(m):
            t, c = items[m]
            sl = pl.ds(base + c * CHUNK, CHUNK)
            wcopies[m] = tuple(
                pltpu.async_copy(bufs[m % DEPTH],
                                 outs[t * n_writes + w].at[sl],
                                 wsems[m % DEPTH])
                for w in range(n_writes))

        def wwait(m):
            for cp in wcopies[m]:
                cp.wait()

        for m in range(min(GLEAD, n)):
            gstart(m)
        for j in range(n):
            gcopies[j].wait()
            wstart(j)
            m = j + GLEAD
            if m < n:
                if m >= DEPTH:
                    wwait(m - DEPTH)
                gstart(m)
        for m in range(max(0, n - DEPTH), n):
            wwait(m)

    return k(idx_flat, *tables_list)


def _dup2_body(x0_ref, x1_ref, o0_ref, o1_ref):
    o0_ref[...] = x0_ref[...]
    o1_ref[...] = x1_ref[...]


def _dup2(x0, x1):
    spec = pl.BlockSpec((DUP_BLOCK, HIDDEN), lambda i: (i, 0))
    out = jax.ShapeDtypeStruct((B, HIDDEN), jnp.float32)
    return pl.pallas_call(
        _dup2_body,
        out_shape=(out, out),
        grid=(B // DUP_BLOCK,),
        in_specs=[spec, spec],
        out_specs=(spec, spec),
    )(x0, x1)


@jax.jit
def _run(idx_flat, W0, W1, W2):
    e0, e1 = _sc_gather(idx_flat, [W0, W1], n_writes=1)
    e2, e3 = _sc_gather(idx_flat, [W2], n_writes=2)
    e5, e4 = _dup2(e0, e1)
    return e0, e1, e2, e3, e4, e5


def kernel(inputs, W0, W1, W2):
    idx = inputs.reshape(-1).astype(jnp.int32)
    outs = _run(idx, W0, W1, W2)
    shp = (*inputs.shape, HIDDEN)
    return tuple(o.reshape(shp) for o in outs)


# speedup vs baseline: 21.5595x; 1.1094x over previous
import functools

import jax
import jax.numpy as jnp
from jax import lax
from jax.experimental import pallas as pl
from jax.experimental.pallas import tpu as pltpu
from jax.experimental.pallas import tpu_sc as plsc

VOCAB = 100000
HIDDEN = 768
B = 4 * 2048
NC, NS = 2, 16
NW = NC * NS
B_PER_W = B // NW
CHUNK = 32
NCHUNK = B_PER_W // CHUNK
DEPTH = 4
GLEAD = 2
DUP_BLOCK = 1024


def _sc_gather(idx_flat, tables_list, n_writes):
    nt = len(tables_list)
    out = jax.ShapeDtypeStruct((B, HIDDEN), jnp.float32)
    mesh = plsc.VectorSubcoreMesh(core_axis_name="c", subcore_axis_name="s")

    @functools.partial(
        pl.kernel,
        out_type=(out,) * (nt * n_writes),
        mesh=mesh,
        scratch_types=[
            *[pltpu.VMEM((CHUNK,), jnp.int32) for _ in range(NCHUNK)],
            *[pltpu.VMEM((CHUNK, HIDDEN), jnp.float32) for _ in range(DEPTH)],
            *[pltpu.SemaphoreType.DMA for _ in range(2 * DEPTH)],
        ],
    )
    def k(idx_hbm, *rest):
        tables = rest[:nt]
        outs = rest[nt:nt + nt * n_writes]
        scratch = rest[nt + nt * n_writes:]
        idxs = scratch[:NCHUNK]
        bufs = scratch[NCHUNK:NCHUNK + DEPTH]
        gsems = scratch[NCHUNK + DEPTH:NCHUNK + 2 * DEPTH]
        wsems = scratch[NCHUNK + 2 * DEPTH:]

        wid = lax.axis_index("s") * NC + lax.axis_index("c")
        base = wid * B_PER_W

        for c in range(NCHUNK):
            pltpu.sync_copy(idx_hbm.at[pl.ds(base + c * CHUNK, CHUNK)], idxs[c])

        items = [(t, c) for t in range(nt) for c in range(NCHUNK)]
        n = len(items)
        gcopies, wcopies = {}, {}

        def gstart(m):
            t, c = items[m]
            gcopies[m] = pltpu.async_copy(
                tables[t].at[idxs[c]], bufs[m % DEPTH], gsems[m % DEPTH])

        def wstart(m):
            t, c = items[m]
            sl = pl.ds(base + c * CHUNK, CHUNK)
            wcopies[m] = tuple(
                pltpu.async_copy(bufs[m % DEPTH],
                                 outs[t * n_writes + w].at[sl],
                                 wsems[m % DEPTH])
                for w in range(n_writes))

        def wwait(m):
            for cp in wcopies[m]:
                cp.wait()

        for m in range(min(GLEAD, n)):
            gstart(m)
        for j in range(n):
            gcopies[j].wait()
            wstart(j)
            m = j + GLEAD
            if m < n:
                if m >= DEPTH:
                    wwait(m - DEPTH)
                gstart(m)
        for m in range(max(0, n - DEPTH), n):
            wwait(m)

    return k(idx_flat, *tables_list)


def _dup2_body(x0_ref, x1_ref, o0_ref, o1_ref):
    o0_ref[...] = x0_ref[...]
    o1_ref[...] = x1_ref[...]


def _dup2(x0, x1):
    spec = pl.BlockSpec((DUP_BLOCK, HIDDEN), lambda i: (i, 0))
    out = jax.ShapeDtypeStruct((B, HIDDEN), jnp.float32)
    return pl.pallas_call(
        _dup2_body,
        out_shape=(out, out),
        grid=(B // DUP_BLOCK,),
        in_specs=[spec, spec],
        out_specs=(spec, spec),
    )(x0, x1)


@jax.jit
def _run(idx_flat, W0, W1, W2):
    e0, e0d, e1, e1d, e2, e2d = _sc_gather(idx_flat, [W0, W1, W2], n_writes=2)
    return e0, e1, e2, e2d, e1d, e0d


def kernel(inputs, W0, W1, W2):
    idx = inputs.reshape(-1).astype(jnp.int32)
    outs = _run(idx, W0, W1, W2)
    shp = (*inputs.shape, HIDDEN)
    return tuple(o.reshape(shp) for o in outs)
